# Initial kernel scaffold; baseline (speedup 1.0000x reference)
import jax
import jax.numpy as jnp
from jax.experimental import pallas as pl


def kernel(input_hidden_states, output_hidden_states, router_hidden_states, W_router, W_down, W_up):
    raise NotImplementedError("write your pallas kernel here")


# speedup vs baseline: 10.8483x; 10.8483x over previous
import jax
import jax.numpy as jnp
from jax import lax
from jax.experimental import pallas as pl
from jax.experimental.pallas import tpu as pltpu

_HID = 2048
_E = 8
_ADIM = 128
_SCALE = 2.0
_TB = 256


def _router_body(r_ref, w_ref, logits_ref, c_ref):
    l = lax.dot_general(r_ref[...], w_ref[...], (((1,), (1,)), ((), ())),
                        preferred_element_type=jnp.float32)
    logits_ref[...] = l
    e_iota = lax.broadcasted_iota(jnp.int32, l.shape, 1)
    m1 = jnp.max(l, axis=1, keepdims=True)
    i1 = jnp.min(jnp.where(l == m1, e_iota, _E), axis=1, keepdims=True)
    lm = jnp.where(e_iota == i1, -jnp.inf, l)
    m2 = jnp.max(lm, axis=1, keepdims=True)
    i2 = jnp.min(jnp.where(lm == m2, e_iota, _E), axis=1, keepdims=True)
    w1 = 1.0 / (1.0 + jnp.exp(m2 - m1))
    w2 = 1.0 - w1
    c = jnp.where(e_iota == i1, w1, 0.0) + jnp.where(e_iota == i2, w2, 0.0)
    c_ref[...] = _SCALE * c


def _adapter_body(x_ref, y_ref, c_ref, wd_ref, wu_ref, m_ref, out_ref):
    xb = x_ref[...].astype(jnp.bfloat16)
    h = lax.dot_general(xb, wd_ref[...], (((1,), (1,)), ((), ())),
                        preferred_element_type=jnp.float32)
    g = 0.5 * h * (1.0 + lax.erf(h * 0.7071067811865476))
    mult = lax.dot_general(c_ref[...], m_ref[...], (((1,), (0,)), ((), ())),
                           preferred_element_type=jnp.float32)
    hs = (g * mult).astype(jnp.bfloat16)
    delta = lax.dot_general(hs, wu_ref[...], (((1,), (0,)), ((), ())),
                            preferred_element_type=jnp.float32)
    out_ref[...] = y_ref[...] + delta


def kernel(input_hidden_states, output_hidden_states, router_hidden_states,
           W_router, W_down, W_up):
    x = input_hidden_states.reshape(-1, _HID)
    y = output_hidden_states.reshape(-1, _HID)
    r = router_hidden_states.reshape(-1, _HID)
    T = x.shape[0]
    grid = T // _TB

    logits, c = pl.pallas_call(
        _router_body,
        grid=(grid,),
        in_specs=[pl.BlockSpec((_TB, _HID), lambda i: (i, 0)),
                  pl.BlockSpec((_E, _HID), lambda i: (0, 0))],
        out_specs=[pl.BlockSpec((_TB, _E), lambda i: (i, 0)),
                   pl.BlockSpec((_TB, _E), lambda i: (i, 0))],
        out_shape=[jax.ShapeDtypeStruct((T, _E), jnp.float32),
                   jax.ShapeDtypeStruct((T, _E), jnp.float32)],
    )(r, W_router)

    wd = W_down.reshape(_E * _ADIM, _HID).astype(jnp.bfloat16)
    wu = W_up.transpose(0, 2, 1).reshape(_E * _ADIM, _HID).astype(jnp.bfloat16)
    m = jnp.repeat(jnp.eye(_E, dtype=jnp.float32), _ADIM, axis=1)

    out = pl.pallas_call(
        _adapter_body,
        grid=(grid,),
        in_specs=[pl.BlockSpec((_TB, _HID), lambda i: (i, 0)),
                  pl.BlockSpec((_TB, _HID), lambda i: (i, 0)),
                  pl.BlockSpec((_TB, _E), lambda i: (i, 0)),
                  pl.BlockSpec((_E * _ADIM, _HID), lambda i: (0, 0)),
                  pl.BlockSpec((_E * _ADIM, _HID), lambda i: (0, 0)),
                  pl.BlockSpec((_E, _E * _ADIM), lambda i: (0, 0))],
        out_specs=pl.BlockSpec((_TB, _HID), lambda i: (i, 0)),
        out_shape=jax.ShapeDtypeStruct((T, _HID), jnp.float32),
    )(x, y, c, wd, wu, m)

    return out.reshape(output_hidden_states.shape), logits
